# baseline (speedup 1.0000x reference)
import functools

import jax
import jax.numpy as jnp
from jax import lax
from jax.experimental import pallas as pl
from jax.experimental.pallas import tpu as pltpu
from jax.experimental.pallas import tpu_sc as plsc
from jax.experimental.layout import Layout, with_layout_constraint

_NUM_WORKERS = 32
_CHUNK = 400
_NBUF = 4
_GLAG = 2


def kernel(x, table):
    batch, seq = x.shape
    _, d_model = table.shape
    n = batch * seq
    idx = x.reshape(n).astype(jnp.int32)

    bpw = batch // _NUM_WORKERS
    per_worker = n // _NUM_WORKERS
    nd16 = d_model // 16
    nb16 = bpw // 16

    mesh = plsc.VectorSubcoreMesh(core_axis_name="c", subcore_axis_name="s")

    @functools.partial(
        pl.kernel,
        out_type=jax.ShapeDtypeStruct((seq, d_model, batch), jnp.float32),
        mesh=mesh,
        compiler_params=pltpu.CompilerParams(
            use_tc_tiling_on_sc=False, needs_layout_passes=False),
        scratch_types=[
            pltpu.VMEM((per_worker,), jnp.int32),
            pltpu.VMEM((2, bpw), jnp.int32),
            pltpu.VMEM((2, bpw, d_model), jnp.float32),
            pltpu.VMEM((2, d_model, bpw), jnp.float32),
            pltpu.SemaphoreType.DMA((2,)),
            pltpu.SemaphoreType.DMA((2,)),
        ],
    )
    def emb(idx_hbm, table_hbm, out_hbm, idx_v, sidx, rows_v, tbuf, gsem, osem):
        wid = lax.axis_index("s") * 2 + lax.axis_index("c")
        b0 = wid * bpw
        base = b0 * seq

        pltpu.sync_copy(idx_hbm.at[pl.ds(base, per_worker)], idx_v)

        iota = lax.iota(jnp.int32, 16)

        def build_sidx(s, r):
            for k in range(nb16):
                g = plsc.load_gather(idx_v, [(iota + 16 * k) * seq + s])
                sidx[r, pl.ds(16 * k, 16)] = g

        def gather_desc(r):
            return pltpu.make_async_copy(
                table_hbm.at[sidx.at[r]], rows_v.at[r], gsem.at[r])

        def write_desc(s, r):
            return pltpu.make_async_copy(
                tbuf.at[r], out_hbm.at[s, :, pl.ds(b0, bpw)], osem.at[r])

        def transpose(r):
            rows = rows_v.at[r]

            @pl.loop(0, d_model)
            def _(d):
                for j in range(nb16):
                    v = plsc.load_gather(
                        rows, [iota + 16 * j, iota * 0 + d])
                    tbuf[r, d, pl.ds(16 * j, 16)] = v

        build_sidx(0, 0)
        gather_desc(0).start()

        @pl.loop(0, seq, step=2)
        def _(g):
            for b in range(2):
                s = g + b
                r = b
                nxt = s + 1

                @pl.when(nxt < seq)
                def _():
                    build_sidx(nxt, 1 - r)
                    gather_desc(1 - r).start()

                gather_desc(r).wait()

                @pl.when(s >= 2)
                def _():
                    write_desc(s - 2, r).wait()

                transpose(r)
                write_desc(s, r).start()

        write_desc(seq - 2, 0).wait()
        write_desc(seq - 1, 1).wait()

    out_t = emb(idx, table)
    return jnp.transpose(out_t, (2, 0, 1))


# speedup vs baseline: 1.6497x; 1.6497x over previous
import functools

import jax
import jax.numpy as jnp
from jax import lax
from jax.experimental import pallas as pl
from jax.experimental.pallas import tpu as pltpu
from jax.experimental.pallas import tpu_sc as plsc

_NUM_WORKERS = 32
_CHUNK = 400
_NBUF = 4
_GLAG = 2


def kernel(x, table):
    batch, seq = x.shape
    _, d_model = table.shape
    n = batch * seq
    idx = x.reshape(n).astype(jnp.int32)

    per_worker = n // _NUM_WORKERS
    n_chunks = per_worker // _CHUNK
    assert per_worker * _NUM_WORKERS == n and n_chunks * _CHUNK == per_worker
    assert n_chunks % _NBUF == 0 and n_chunks > 2 * _NBUF

    mesh = plsc.VectorSubcoreMesh(core_axis_name="c", subcore_axis_name="s")

    @functools.partial(
        pl.kernel,
        out_type=jax.ShapeDtypeStruct((n, d_model), jnp.float32),
        mesh=mesh,
        compiler_params=pltpu.CompilerParams(use_tc_tiling_on_sc=False),
        scratch_types=[
            pltpu.VMEM((per_worker,), jnp.int32),
            pltpu.VMEM((_NBUF, _CHUNK, d_model), jnp.float32),
            pltpu.SemaphoreType.DMA((_NBUF,)),
            pltpu.SemaphoreType.DMA((_NBUF,)),
        ],
    )
    def emb(idx_hbm, table_hbm, out_hbm, idx_v, rows_v, gsem, osem):
        wid = lax.axis_index("s") * 2 + lax.axis_index("c")
        base = wid * per_worker

        def gather_desc(c, b):
            idx_slice = idx_v.at[pl.ds(c * _CHUNK, _CHUNK)]
            return pltpu.make_async_copy(
                table_hbm.at[idx_slice], rows_v.at[b], gsem.at[b])

        def write_desc(c, b):
            dst = out_hbm.at[pl.ds(base + c * _CHUNK, _CHUNK)]
            return pltpu.make_async_copy(rows_v.at[b], dst, osem.at[b])

        pltpu.sync_copy(idx_hbm.at[pl.ds(base, per_worker)], idx_v)

        for c in range(_GLAG):
            gather_desc(c, c % _NBUF).start()

        for w in range(_GLAG):
            gather_desc(w, w % _NBUF).wait()
            write_desc(w, w % _NBUF).start()
            ng = w + _GLAG
            gather_desc(ng, ng % _NBUF).start()

        @pl.loop(_GLAG, n_chunks - _GLAG, step=_NBUF)
        def _(g):
            for b in range(_NBUF):
                w = g + b
                bw = (_GLAG + b) % _NBUF
                gather_desc(w, bw).wait()
                write_desc(w, bw).start()
                ng = w + _GLAG
                bg = (2 * _GLAG + b) % _NBUF
                write_desc(ng - _NBUF, bg).wait()
                gather_desc(ng, bg).start()

        for w in range(n_chunks - _GLAG, n_chunks):
            gather_desc(w, w % _NBUF).wait()
            write_desc(w, w % _NBUF).start()

        for w in range(n_chunks - _NBUF, n_chunks):
            write_desc(w, w % _NBUF).wait()

    out = emb(idx, table)
    return out.reshape(batch, seq, d_model)
